# baseline (speedup 1.0000x reference)
import functools

import jax
import jax.numpy as jnp
from jax.experimental import pallas as pl
from jax.experimental.pallas import tpu as pltpu

MAX_MASKS = 5
RING_K = 7
RING_R = RING_K // 2


def _topk_body(scores_ref, idx_ref, *, m):
    s = scores_ref[...]
    b_dim, k_dim = s.shape
    lane = jax.lax.broadcasted_iota(jnp.int32, s.shape, 1)
    cols = []
    for _ in range(m):
        mx = jnp.max(s, axis=1, keepdims=True)
        amx = jnp.min(jnp.where(s == mx, lane, k_dim), axis=1, keepdims=True)
        cols.append(amx.astype(jnp.int32))
        s = jnp.where(lane == amx, -jnp.inf, s)
    idx_ref[...] = jnp.concatenate(cols, axis=1)


def _topk(mask_scores, m):
    b_dim, k_dim = mask_scores.shape
    return pl.pallas_call(
        functools.partial(_topk_body, m=m),
        out_shape=jax.ShapeDtypeStruct((b_dim, m), jnp.int32),
    )(mask_scores)


def _dilate_1d(p, axis, radius):
    neg = jnp.full_like(p, -jnp.inf)

    def shifted_max(x, d):
        if axis == 0:
            up = jnp.concatenate([x[d:, :], neg[:d, :]], axis=0)
            dn = jnp.concatenate([neg[:d, :], x[:-d, :]], axis=0)
        else:
            up = jnp.concatenate([x[:, d:], neg[:, :d]], axis=1)
            dn = jnp.concatenate([neg[:, :d], x[:, :-d]], axis=1)
        return jnp.maximum(x, jnp.maximum(up, dn))

    assert radius == 3
    return shifted_max(shifted_max(p, 1), 2)


def _plane_body(idx_ref, ml_ref, wpl_ref):
    x = ml_ref[0, 0]
    h, w = x.shape
    p = jax.nn.sigmoid(x)
    dil = _dilate_1d(_dilate_1d(p, 0, RING_R), 1, RING_R)
    ring = jnp.maximum(dil - p, 0.0)
    sp = jnp.sum(p)
    wpl_ref[0, 0, 0] = p / jnp.maximum(sp, 1e-6)
    sr = jnp.sum(ring)
    empty = (sr == 0.0).astype(jnp.float32)
    ring = ring + empty * 1e-4
    sr2 = sr + empty * (1e-4 * h * w)
    wpl_ref[0, 1, 0] = ring / jnp.maximum(sr2, 1e-6)


def _planes(mask_logits, idx, m):
    b_dim, k_dim, h, w = mask_logits.shape
    grid_spec = pltpu.PrefetchScalarGridSpec(
        num_scalar_prefetch=1,
        grid=(b_dim, m),
        in_specs=[pl.BlockSpec((1, 1, h, w),
                               lambda b, j, idx_ref: (b, idx_ref[b, j], 0, 0))],
        out_specs=[pl.BlockSpec((1, 2, 1, h, w),
                                lambda b, j, idx_ref: (b, 0, j, 0, 0))],
    )
    return pl.pallas_call(
        _plane_body,
        grid_spec=grid_spec,
        out_shape=[jax.ShapeDtypeStruct((b_dim, 2, m, h, w), jnp.float32)],
        compiler_params=pltpu.CompilerParams(
            dimension_semantics=("arbitrary", "arbitrary")),
    )(idx, mask_logits)[0]


def _pool_body(wpl_ref, pf_ref, wev_ref, bev_ref, out_ref, acc_ref, tpf_ref,
               *, m, nchunk, hchunk):
    c = pl.program_id(1)

    @pl.when(c == 0)
    def _():
        acc_ref[...] = jnp.zeros_like(acc_ref)

    tpf_ref[...] = jnp.transpose(pf_ref[0], (1, 0, 2))
    s = jnp.zeros_like(acc_ref)
    for h in range(hchunk):
        w2 = wpl_ref[0][:, :, h, :].reshape(2 * m, -1)
        s = s + jax.lax.dot_general(
            w2, tpf_ref[h], (((1,), (1,)), ((), ())),
            preferred_element_type=jnp.float32)
    acc_ref[...] += s

    @pl.when(c == nchunk - 1)
    def _():
        ev = jax.lax.dot_general(
            acc_ref[...], wev_ref[...], (((1,), (1,)), ((), ())),
            preferred_element_type=jnp.float32)
        out_ref[0] = ev + bev_ref[...]


def _pool(wpl, pixel_feat, w_ev, b_ev2, m, hchunk):
    b_dim, c_dim, h, w = pixel_feat.shape
    d_dim = w_ev.shape[0]
    nchunk = h // hchunk
    return pl.pallas_call(
        functools.partial(_pool_body, m=m, nchunk=nchunk, hchunk=hchunk),
        grid=(b_dim, nchunk),
        in_specs=[
            pl.BlockSpec((1, 2, m, hchunk, w), lambda b, c: (b, 0, 0, c, 0)),
            pl.BlockSpec((1, c_dim, hchunk, w), lambda b, c: (b, 0, c, 0)),
            pl.BlockSpec((d_dim, c_dim), lambda b, c: (0, 0)),
            pl.BlockSpec((1, d_dim), lambda b, c: (0, 0)),
        ],
        out_specs=[pl.BlockSpec((1, 2 * m, d_dim), lambda b, c: (b, 0, 0))],
        out_shape=[jax.ShapeDtypeStruct((b_dim, 2 * m, d_dim), jnp.float32)],
        scratch_shapes=[
            pltpu.VMEM((2 * m, c_dim), jnp.float32),
            pltpu.VMEM((hchunk, c_dim, w), jnp.float32),
        ],
        compiler_params=pltpu.CompilerParams(
            dimension_semantics=("arbitrary", "arbitrary")),
    )(wpl, pixel_feat, w_ev, b_ev2)[0]


def kernel(mask_logits, pixel_feat, mask_scores, W_ev, b_ev):
    b_dim, k_dim, h, w = mask_logits.shape
    m = min(MAX_MASKS, k_dim)

    idx = _topk(mask_scores, m)
    wpl = _planes(mask_logits, idx, m)
    ev = _pool(wpl, pixel_feat, W_ev, b_ev.reshape(1, -1), m, hchunk=56)
    return ev


# speedup vs baseline: 1.1974x; 1.1974x over previous
import functools

import jax
import jax.numpy as jnp
from jax.experimental import pallas as pl
from jax.experimental.pallas import tpu as pltpu

MAX_MASKS = 5
RING_K = 7
RING_R = RING_K // 2


def _topk_body(scores_ref, idx_ref, *, m):
    s = scores_ref[...]
    b_dim, k_dim = s.shape
    lane = jax.lax.broadcasted_iota(jnp.int32, s.shape, 1)
    cols = []
    for _ in range(m):
        mx = jnp.max(s, axis=1, keepdims=True)
        amx = jnp.min(jnp.where(s == mx, lane, k_dim), axis=1, keepdims=True)
        cols.append(amx.astype(jnp.int32))
        s = jnp.where(lane == amx, -jnp.inf, s)
    idx_ref[...] = jnp.concatenate(cols, axis=1)


def _topk(mask_scores, m):
    b_dim, k_dim = mask_scores.shape
    return pl.pallas_call(
        functools.partial(_topk_body, m=m),
        out_shape=jax.ShapeDtypeStruct((b_dim, m), jnp.int32),
    )(mask_scores)


def _dilate_1d(p, axis, radius):
    neg = jnp.full_like(p, -jnp.inf)

    def shifted_max(x, d):
        if axis == 0:
            up = jnp.concatenate([x[d:, :], neg[:d, :]], axis=0)
            dn = jnp.concatenate([neg[:d, :], x[:-d, :]], axis=0)
        else:
            up = jnp.concatenate([x[:, d:], neg[:, :d]], axis=1)
            dn = jnp.concatenate([neg[:, :d], x[:, :-d]], axis=1)
        return jnp.maximum(x, jnp.maximum(up, dn))

    assert radius == 3
    return shifted_max(shifted_max(p, 1), 2)


def _fused_body(idx_ref, *refs, m, nchunk, hchunk):
    ml_refs = refs[:m]
    pf_ref, wev_ref, bev_ref, out_ref, wpl_ref, acc_ref, tpf_ref = refs[m:]
    c = pl.program_id(1)

    @pl.when(c == 0)
    def _():
        acc_ref[...] = jnp.zeros_like(acc_ref)
        for j, mlr in enumerate(ml_refs):
            x = mlr[0, 0]
            h, w = x.shape
            p = jax.nn.sigmoid(x)
            dil = _dilate_1d(_dilate_1d(p, 0, RING_R), 1, RING_R)
            ring = jnp.maximum(dil - p, 0.0)
            sp = jnp.sum(p)
            wpl_ref[j] = p / jnp.maximum(sp, 1e-6)
            sr = jnp.sum(ring)
            empty = (sr == 0.0).astype(jnp.float32)
            ring = ring + empty * 1e-4
            sr2 = sr + empty * (1e-4 * h * w)
            wpl_ref[m + j] = ring / jnp.maximum(sr2, 1e-6)

    base = c * hchunk
    tpf_ref[...] = jnp.transpose(pf_ref[0], (1, 0, 2))
    s = jnp.zeros_like(acc_ref)
    for h in range(hchunk):
        w2 = wpl_ref[:, base + h, :]
        s = s + jax.lax.dot_general(
            w2, tpf_ref[h], (((1,), (1,)), ((), ())),
            preferred_element_type=jnp.float32)
    acc_ref[...] += s

    @pl.when(c == nchunk - 1)
    def _():
        ev = jax.lax.dot_general(
            acc_ref[...], wev_ref[...], (((1,), (1,)), ((), ())),
            preferred_element_type=jnp.float32)
        out_ref[0] = ev + bev_ref[...]


def _fused(mask_logits, idx, pixel_feat, w_ev, b_ev2, m, hchunk):
    b_dim, k_dim, h, w = mask_logits.shape
    c_dim = pixel_feat.shape[1]
    d_dim = w_ev.shape[0]
    nchunk = h // hchunk

    def mask_map(j):
        return lambda b, c, idx_ref: (b, idx_ref[b, j], 0, 0)

    in_specs = [pl.BlockSpec((1, 1, h, w), mask_map(j)) for j in range(m)]
    in_specs += [
        pl.BlockSpec((1, c_dim, hchunk, w), lambda b, c, idx_ref: (b, 0, c, 0)),
        pl.BlockSpec((d_dim, c_dim), lambda b, c, idx_ref: (0, 0)),
        pl.BlockSpec((1, d_dim), lambda b, c, idx_ref: (0, 0)),
    ]
    grid_spec = pltpu.PrefetchScalarGridSpec(
        num_scalar_prefetch=1,
        grid=(b_dim, nchunk),
        in_specs=in_specs,
        out_specs=[pl.BlockSpec((1, 2 * m, d_dim),
                                lambda b, c, idx_ref: (b, 0, 0))],
        scratch_shapes=[
            pltpu.VMEM((2 * m, h, w), jnp.float32),
            pltpu.VMEM((2 * m, c_dim), jnp.float32),
            pltpu.VMEM((hchunk, c_dim, w), jnp.float32),
        ],
    )
    return pl.pallas_call(
        functools.partial(_fused_body, m=m, nchunk=nchunk, hchunk=hchunk),
        grid_spec=grid_spec,
        out_shape=[jax.ShapeDtypeStruct((b_dim, 2 * m, d_dim), jnp.float32)],
        compiler_params=pltpu.CompilerParams(
            dimension_semantics=("parallel", "arbitrary")),
    )(idx, *([mask_logits] * m), pixel_feat, w_ev, b_ev2)[0]


def kernel(mask_logits, pixel_feat, mask_scores, W_ev, b_ev):
    b_dim, k_dim, h, w = mask_logits.shape
    m = min(MAX_MASKS, k_dim)

    idx = _topk(mask_scores, m)
    ev = _fused(mask_logits, idx, pixel_feat, W_ev, b_ev.reshape(1, -1),
                m, hchunk=56)
    return ev
